# baseline (speedup 1.0000x reference)
import functools

import jax
import jax.numpy as jnp
from jax import lax
from jax.experimental import pallas as pl
from jax.experimental.pallas import tpu as pltpu
from jax.experimental.pallas import tpu_sc as plsc

_V = 1_000_000
_D = 32
_B = 4096
_L = 50
_VMAIN = 999_936
_TAIL = _V - _VMAIN
_PIECE = 768
_NPIECE = _VMAIN // _PIECE
_SRP = _PIECE // 4
_NSR = _VMAIN // 4


def _build_transpose(nc: int, ns: int):
    nw = nc * ns
    kmax = 42
    mesh = plsc.VectorSubcoreMesh(core_axis_name="c", subcore_axis_name="s")

    @functools.partial(
        pl.kernel,
        mesh=mesh,
        compiler_params=pltpu.CompilerParams(needs_layout_passes=False),
        out_type=jax.ShapeDtypeStruct((_NSR, 128), jnp.float32),
        scratch_types=[
            pltpu.VMEM((_D, _PIECE), jnp.float32),
            pltpu.VMEM((_D, _PIECE), jnp.float32),
            pltpu.VMEM((_SRP, 128), jnp.float32),
            pltpu.VMEM((_SRP, 128), jnp.float32),
            pltpu.SemaphoreType.DMA,
            pltpu.SemaphoreType.DMA,
            pltpu.SemaphoreType.DMA,
            pltpu.SemaphoreType.DMA,
        ],
    )
    def ker(tab_hbm, lin_hbm, in0, in1, ot0, ot1, is0, is1, os0, os1):
        cid = lax.axis_index("c")
        sid = lax.axis_index("s")
        wid = sid * nc + cid
        inb = (in0, in1)
        otb = (ot0, ot1)
        isem = (is0, is1)
        osem = (os0, os1)

        def piece_of(k):
            return lax.rem(wid + 32 * k, _NPIECE)

        def fire_streams(k, slot):
            p = piece_of(k)
            for t in range(4):
                pltpu.async_copy(
                    tab_hbm.at[t, :, pl.ds(p * _PIECE, _PIECE)],
                    inb[slot].at[pl.ds(8 * t, 8), :],
                    isem[slot],
                )

        def wait_streams(slot):
            for _ in range(4):
                pltpu.make_async_copy(
                    tab_hbm.at[0, :, pl.ds(0, _PIECE)],
                    inb[slot].at[pl.ds(0, 8), :],
                    isem[slot],
                ).wait()

        def wait_store(slot):
            pltpu.make_async_copy(
                otb[slot], lin_hbm.at[pl.ds(0, _SRP), :], osem[slot]
            ).wait()

        fire_streams(0, 0)
        fire_streams(1, 1)

        f_lo = lax.iota(jnp.int32, 16)
        f_hi = f_lo + 16

        def do_piece(k, slot, first):
            @pl.when(jnp.logical_not(first))
            def _():
                wait_store(slot)

            wait_streams(slot)

            @plsc.parallel_loop(0, _SRP, unroll=16)
            def _(sr):
                for v in range(8):
                    fv = f_lo if v % 2 == 0 else f_hi
                    rl = jnp.full((16,), 4 * sr + v // 2, jnp.int32)
                    vals = plsc.load_gather(inb[slot], [fv, rl])
                    otb[slot][sr, pl.ds(16 * v, 16)] = vals
            p = piece_of(k)
            pltpu.async_copy(
                otb[slot], lin_hbm.at[pl.ds(p * _SRP, _SRP), :], osem[slot]
            )
            fire_streams(lax.rem(k + 2, kmax), slot)

        def pair(q, carry):
            do_piece(2 * q, 0, q == 0)
            do_piece(2 * q + 1, 1, q == 0)
            return carry

        lax.fori_loop(0, kmax // 2, pair, 0)
        wait_streams(0)
        wait_streams(1)
        wait_store(0)
        wait_store(1)

    return ker


def _build_lookup(nc: int, ns: int):
    bpw = _B // (nc * ns)
    mesh = plsc.VectorSubcoreMesh(core_axis_name="c", subcore_axis_name="s")

    @functools.partial(
        pl.kernel,
        mesh=mesh,
        compiler_params=pltpu.CompilerParams(needs_layout_passes=False),
        out_type=jax.ShapeDtypeStruct((_L, 4, 8, _B), jnp.float32),
        scratch_types=[
            pltpu.VMEM((_L, bpw), jnp.int32),
            pltpu.VMEM((_L, bpw), jnp.int32),
            pltpu.VMEM((bpw, 128), jnp.float32),
            pltpu.VMEM((bpw, 128), jnp.float32),
            pltpu.VMEM((2, 4, 8, bpw), jnp.float32),
            pltpu.VMEM((_TAIL, _D), jnp.float32),
            pltpu.SemaphoreType.DMA,
            pltpu.SemaphoreType.DMA,
        ],
    )
    def ker(lin_hbm, txt_hbm, tail_hbm, out_hbm, idx_v, sidx_v,
            sl0, sl1, ebuf, tail_v, gsem, osem):
        cid = lax.axis_index("c")
        sid = lax.axis_index("s")
        boff = cid * (ns * bpw) + sid * bpw
        slabs = (sl0, sl1)

        pltpu.sync_copy(txt_hbm.at[:, pl.ds(boff, bpw)], idx_v)
        pltpu.sync_copy(tail_hbm, tail_v)

        def mk_sidx(li, carry):
            for g in range(bpw // 16):
                iv = idx_v[li, pl.ds(16 * g, 16)]
                sidx_v[li, pl.ds(16 * g, 16)] = jnp.minimum(
                    lax.shift_right_logical(iv, 2), _NSR - 1
                )
            return carry

        lax.fori_loop(0, _L, mk_sidx, 0)

        def fire_gather(li, slot):
            pltpu.async_copy(lin_hbm.at[sidx_v.at[li]], slabs[slot], gsem)

        def wait_gather(slot):
            pltpu.make_async_copy(
                lin_hbm.at[pl.ds(0, bpw)], slabs[slot], gsem
            ).wait()

        def wait_ostores(par):
            pltpu.make_async_copy(
                ebuf.at[par], out_hbm.at[0, :, :, pl.ds(boff, bpw)], osem
            ).wait()

        fire_gather(0, 0)
        lanes = lax.iota(jnp.int32, 16)

        def do_l(li, slot, par, first):
            @pl.when(jnp.logical_not(first))
            def _():
                wait_ostores(par)

            wait_gather(slot)
            fire_gather(lax.rem(li + 1, _L), 1 - slot)
            slab = slabs[slot]

            @plsc.parallel_loop(0, bpw // 16, unroll=4)
            def _(g):
                rows = lanes + 16 * g
                iv = idx_v[li, pl.ds(16 * g, 16)]
                off = (iv & 3) * 32
                for t in range(4):
                    for f in range(8):
                        vals = plsc.load_gather(slab, [rows, off + (8 * t + f)])
                        ebuf[par, t, f, pl.ds(16 * g, 16)] = vals

            mx0 = jnp.max(idx_v[li, pl.ds(0, 16)])

            def fold(g, m):
                return jnp.maximum(m, jnp.max(idx_v[li, pl.ds(16 * g, 16)]))

            mx = lax.fori_loop(1, bpw // 16, fold, mx0)

            @pl.when(mx >= _VMAIN)
            def _():
                def fixg(g, carry):
                    iv = idx_v[li, pl.ds(16 * g, 16)]
                    m = iv >= _VMAIN
                    tr = jnp.maximum(iv - _VMAIN, 0)
                    for t in range(4):
                        for f in range(8):
                            tv = plsc.load_gather(
                                tail_v, [tr, jnp.full((16,), 8 * t + f,
                                                      jnp.int32)]
                            )
                            cur = ebuf[par, t, f, pl.ds(16 * g, 16)]
                            ebuf[par, t, f, pl.ds(16 * g, 16)] = jnp.where(
                                m, tv, cur
                            )
                    return carry

                lax.fori_loop(0, bpw // 16, fixg, 0)

            for t in range(4):
                pltpu.async_copy(
                    ebuf.at[par, t],
                    out_hbm.at[li, t, :, pl.ds(boff, bpw)],
                    osem,
                )

        def pair(q, carry):
            do_l(2 * q, 0, 0, q == 0)
            do_l(2 * q + 1, 1, 1, q == 0)
            return carry

        lax.fori_loop(0, _L // 2, pair, 0)
        wait_gather(0)
        wait_ostores(0)
        wait_ostores(1)

    return ker


def kernel(text, table):
    b, l = text.shape
    v, d = table.shape
    assert (b, l, v, d) == (_B, _L, _V, _D)

    info = plsc.get_sparse_core_info()
    nc, ns = info.num_cores, info.num_subcores

    tab = table.T.reshape(d // 8, 8, v)
    txt = text.T
    tail = table[_VMAIN:]

    lin = _build_transpose(nc, ns)(tab)
    out4 = _build_lookup(nc, ns)(lin, txt, tail)
    return out4.reshape(_L, _D, _B).transpose(2, 0, 1)


# speedup vs baseline: 1.0283x; 1.0283x over previous
import functools

import jax
import jax.numpy as jnp
from jax import lax
from jax.experimental import pallas as pl
from jax.experimental.pallas import tpu as pltpu
from jax.experimental.pallas import tpu_sc as plsc

_V = 1_000_000
_D = 32
_B = 4096
_L = 50
_VMAIN = 999_936
_TAIL = _V - _VMAIN
_PIECE = 768
_NPIECE = _VMAIN // _PIECE
_SRP = _PIECE // 4
_NSR = _VMAIN // 4


def _build_transpose(nc: int, ns: int):
    nw = nc * ns
    kmax = 42
    mesh = plsc.VectorSubcoreMesh(core_axis_name="c", subcore_axis_name="s")

    @functools.partial(
        pl.kernel,
        mesh=mesh,
        compiler_params=pltpu.CompilerParams(needs_layout_passes=False),
        out_type=jax.ShapeDtypeStruct((_NSR, 128), jnp.float32),
        scratch_types=[
            pltpu.VMEM((_D, _PIECE), jnp.float32),
            pltpu.VMEM((_D, _PIECE), jnp.float32),
            pltpu.VMEM((_SRP, 128), jnp.float32),
            pltpu.VMEM((_SRP, 128), jnp.float32),
            pltpu.SemaphoreType.DMA,
            pltpu.SemaphoreType.DMA,
            pltpu.SemaphoreType.DMA,
            pltpu.SemaphoreType.DMA,
        ],
    )
    def ker(tab_hbm, lin_hbm, in0, in1, ot0, ot1, is0, is1, os0, os1):
        cid = lax.axis_index("c")
        sid = lax.axis_index("s")
        wid = sid * nc + cid
        inb = (in0, in1)
        otb = (ot0, ot1)
        isem = (is0, is1)
        osem = (os0, os1)

        def piece_of(k):
            return lax.rem(wid + 32 * k, _NPIECE)

        def fire_streams(k, slot):
            p = piece_of(k)
            for t in range(4):
                pltpu.async_copy(
                    tab_hbm.at[t, :, pl.ds(p * _PIECE, _PIECE)],
                    inb[slot].at[pl.ds(8 * t, 8), :],
                    isem[slot],
                )

        def wait_streams(slot):
            for _ in range(4):
                pltpu.make_async_copy(
                    tab_hbm.at[0, :, pl.ds(0, _PIECE)],
                    inb[slot].at[pl.ds(0, 8), :],
                    isem[slot],
                ).wait()

        def wait_store(slot):
            pltpu.make_async_copy(
                otb[slot], lin_hbm.at[pl.ds(0, _SRP), :], osem[slot]
            ).wait()

        fire_streams(0, 0)
        fire_streams(1, 1)

        f_lo = lax.iota(jnp.int32, 16)
        f_hi = f_lo + 16

        def do_piece(k, slot, first):
            @pl.when(jnp.logical_not(first))
            def _():
                wait_store(slot)

            wait_streams(slot)

            @plsc.parallel_loop(0, _SRP, unroll=4)
            def _(sr):
                for v in range(8):
                    fv = f_lo if v % 2 == 0 else f_hi
                    rl = jnp.full((16,), 4 * sr + v // 2, jnp.int32)
                    vals = plsc.load_gather(inb[slot], [fv, rl])
                    otb[slot][sr, pl.ds(16 * v, 16)] = vals
            p = piece_of(k)
            pltpu.async_copy(
                otb[slot], lin_hbm.at[pl.ds(p * _SRP, _SRP), :], osem[slot]
            )
            fire_streams(lax.rem(k + 2, kmax), slot)

        def pair(q, carry):
            do_piece(2 * q, 0, q == 0)
            do_piece(2 * q + 1, 1, q == 0)
            return carry

        lax.fori_loop(0, kmax // 2, pair, 0)
        wait_streams(0)
        wait_streams(1)
        wait_store(0)
        wait_store(1)

    return ker


def _build_lookup(nc: int, ns: int):
    bpw = _B // (nc * ns)
    mesh = plsc.VectorSubcoreMesh(core_axis_name="c", subcore_axis_name="s")

    @functools.partial(
        pl.kernel,
        mesh=mesh,
        compiler_params=pltpu.CompilerParams(needs_layout_passes=False),
        out_type=jax.ShapeDtypeStruct((_L, 4, 8, _B), jnp.float32),
        scratch_types=[
            pltpu.VMEM((_L, bpw), jnp.int32),
            pltpu.VMEM((_L, bpw), jnp.int32),
            pltpu.VMEM((bpw, 128), jnp.float32),
            pltpu.VMEM((bpw, 128), jnp.float32),
            pltpu.VMEM((2, 4, 8, bpw), jnp.float32),
            pltpu.VMEM((_TAIL, _D), jnp.float32),
            pltpu.SemaphoreType.DMA,
            pltpu.SemaphoreType.DMA,
        ],
    )
    def ker(lin_hbm, txt_hbm, tail_hbm, out_hbm, idx_v, sidx_v,
            sl0, sl1, ebuf, tail_v, gsem, osem):
        cid = lax.axis_index("c")
        sid = lax.axis_index("s")
        boff = cid * (ns * bpw) + sid * bpw
        slabs = (sl0, sl1)

        pltpu.sync_copy(txt_hbm.at[:, pl.ds(boff, bpw)], idx_v)
        pltpu.sync_copy(tail_hbm, tail_v)

        def mk_sidx(li, carry):
            for g in range(bpw // 16):
                iv = idx_v[li, pl.ds(16 * g, 16)]
                sidx_v[li, pl.ds(16 * g, 16)] = jnp.minimum(
                    lax.shift_right_logical(iv, 2), _NSR - 1
                )
            return carry

        lax.fori_loop(0, _L, mk_sidx, 0)

        def fire_gather(li, slot):
            pltpu.async_copy(lin_hbm.at[sidx_v.at[li]], slabs[slot], gsem)

        def wait_gather(slot):
            pltpu.make_async_copy(
                lin_hbm.at[pl.ds(0, bpw)], slabs[slot], gsem
            ).wait()

        def wait_ostores(par):
            pltpu.make_async_copy(
                ebuf.at[par], out_hbm.at[0, :, :, pl.ds(boff, bpw)], osem
            ).wait()

        fire_gather(0, 0)
        lanes = lax.iota(jnp.int32, 16)

        def do_l(li, slot, par, first):
            @pl.when(jnp.logical_not(first))
            def _():
                wait_ostores(par)

            wait_gather(slot)
            fire_gather(lax.rem(li + 1, _L), 1 - slot)
            slab = slabs[slot]

            @plsc.parallel_loop(0, bpw // 16, unroll=2)
            def _(g):
                rows = lanes + 16 * g
                iv = idx_v[li, pl.ds(16 * g, 16)]
                off = (iv & 3) * 32
                for t in range(4):
                    for f in range(8):
                        vals = plsc.load_gather(slab, [rows, off + (8 * t + f)])
                        ebuf[par, t, f, pl.ds(16 * g, 16)] = vals

            mx0 = jnp.max(idx_v[li, pl.ds(0, 16)])

            def fold(g, m):
                return jnp.maximum(m, jnp.max(idx_v[li, pl.ds(16 * g, 16)]))

            mx = lax.fori_loop(1, bpw // 16, fold, mx0)

            @pl.when(mx >= _VMAIN)
            def _():
                def fixg(g, carry):
                    iv = idx_v[li, pl.ds(16 * g, 16)]
                    m = iv >= _VMAIN
                    tr = jnp.maximum(iv - _VMAIN, 0)
                    for t in range(4):
                        for f in range(8):
                            tv = plsc.load_gather(
                                tail_v, [tr, jnp.full((16,), 8 * t + f,
                                                      jnp.int32)]
                            )
                            cur = ebuf[par, t, f, pl.ds(16 * g, 16)]
                            ebuf[par, t, f, pl.ds(16 * g, 16)] = jnp.where(
                                m, tv, cur
                            )
                    return carry

                lax.fori_loop(0, bpw // 16, fixg, 0)

            for t in range(4):
                pltpu.async_copy(
                    ebuf.at[par, t],
                    out_hbm.at[li, t, :, pl.ds(boff, bpw)],
                    osem,
                )

        def pair(q, carry):
            do_l(2 * q, 0, 0, q == 0)
            do_l(2 * q + 1, 1, 1, q == 0)
            return carry

        lax.fori_loop(0, _L // 2, pair, 0)
        wait_gather(0)
        wait_ostores(0)
        wait_ostores(1)

    return ker


def kernel(text, table):
    b, l = text.shape
    v, d = table.shape
    assert (b, l, v, d) == (_B, _L, _V, _D)

    info = plsc.get_sparse_core_info()
    nc, ns = info.num_cores, info.num_subcores

    tab = table.T.reshape(d // 8, 8, v)
    txt = text.T
    tail = table[_VMAIN:]

    lin = _build_transpose(nc, ns)(tab)
    out4 = _build_lookup(nc, ns)(lin, txt, tail)
    return out4.reshape(_L, _D, _B).transpose(2, 0, 1)
